# baseline (speedup 1.0000x reference)
import jax
import jax.numpy as jnp
from jax.experimental import pallas as pl
from jax.experimental.pallas import tpu as pltpu


def _enc_block(x_ref, w_ref, b_ref, o_ref, *, K: int, S: int):
    xs = x_ref[0]
    acc = xs[:, 0:S]
    for k in range(1, K):
        acc = acc + xs[:, k * S:(k + 1) * S]
    m = acc * (1.0 / K)
    z = jnp.dot(m, w_ref[...], preferred_element_type=jnp.float32)
    z = z + b_ref[...]
    o_ref[...] = jnp.tile(z, (1, K))[None]


def kernel(x, W, b):
    B, T, K, S = x.shape
    Z = W.shape[1]
    TT = 100
    grid = (B, T // TT)
    import functools
    body = functools.partial(_enc_block, K=K, S=S)
    out = pl.pallas_call(
        body,
        grid=grid,
        in_specs=[
            pl.BlockSpec((1, TT, K * S), lambda i, j: (i, j, 0)),
            pl.BlockSpec((S, Z), lambda i, j: (0, 0)),
            pl.BlockSpec((1, Z), lambda i, j: (0, 0)),
        ],
        out_specs=pl.BlockSpec((1, TT, K * Z), lambda i, j: (i, j, 0)),
        out_shape=jax.ShapeDtypeStruct((B, T, K * Z), jnp.float32),
        compiler_params=pltpu.CompilerParams(
            dimension_semantics=("parallel", "parallel")),
    )(x.reshape(B, T, K * S), W, b.reshape(1, Z))
    return out.reshape(B, T, K, Z)


# speedup vs baseline: 1.0292x; 1.0292x over previous
import functools

import jax
import jax.numpy as jnp
from jax import lax
from jax.experimental import pallas as pl
from jax.experimental.pallas import tpu as pltpu
from jax.experimental.pallas import tpu_sc as plsc

_NC, _NS, _L = 2, 16, 16


def _sc_mean(x_hbm, m_hbm, buf, mbuf, *, T, K, S, C):
    wid = lax.axis_index("s") * _NC + lax.axis_index("c")

    def chunk_body(i, _):
        t0 = i * C
        pltpu.sync_copy(x_hbm.at[wid, pl.ds(t0, C)], buf)

        def t_body(t, _):
            for c in range(S // _L):
                acc = buf[t, 0, pl.ds(c * _L, _L)]
                for k in range(1, K):
                    acc = acc + buf[t, k, pl.ds(c * _L, _L)]
                mbuf[t0 + t, pl.ds(c * _L, _L)] = acc * (1.0 / K)
            return 0

        lax.fori_loop(0, C, t_body, 0)
        return 0

    lax.fori_loop(0, T // C, chunk_body, 0)
    pltpu.sync_copy(mbuf, m_hbm.at[wid])


def _tc_proj(m_ref, w_ref, b_ref, o_ref, *, K):
    m = m_ref[0]
    z = jnp.dot(m, w_ref[...], preferred_element_type=jnp.float32)
    z = z + b_ref[...]
    o_ref[...] = jnp.tile(z, (1, K))[None]


def kernel(x, W, b):
    B, T, K, S = x.shape
    Z = W.shape[1]
    C = 10

    mesh = plsc.VectorSubcoreMesh(core_axis_name="c", subcore_axis_name="s")
    sc_mean = pl.kernel(
        functools.partial(_sc_mean, T=T, K=K, S=S, C=C),
        out_type=jax.ShapeDtypeStruct((B, T, S), jnp.float32),
        mesh=mesh,
        scratch_types=[
            pltpu.VMEM((C, K, S), jnp.float32),
            pltpu.VMEM((T, S), jnp.float32),
        ],
    )
    m = sc_mean(x)

    TT = T
    out = pl.pallas_call(
        functools.partial(_tc_proj, K=K),
        grid=(B, T // TT),
        in_specs=[
            pl.BlockSpec((1, TT, S), lambda i, j: (i, j, 0)),
            pl.BlockSpec((S, Z), lambda i, j: (0, 0)),
            pl.BlockSpec((1, Z), lambda i, j: (0, 0)),
        ],
        out_specs=pl.BlockSpec((1, TT, K * Z), lambda i, j: (i, j, 0)),
        out_shape=jax.ShapeDtypeStruct((B, T, K * Z), jnp.float32),
        compiler_params=pltpu.CompilerParams(
            dimension_semantics=("parallel", "parallel")),
    )(m, W, b.reshape(1, Z))
    return out.reshape(B, T, K, Z)


# speedup vs baseline: 1.2595x; 1.2238x over previous
import functools

import jax
import jax.numpy as jnp
from jax import lax
from jax.experimental import pallas as pl
from jax.experimental.pallas import tpu as pltpu
from jax.experimental.pallas import tpu_sc as plsc

_NC, _NS, _L = 2, 16, 16


def _sc_mean(x_hbm, m_hbm, buf0, buf1, mbuf, sem0, sem1, *, T, K, S, C):
    wid = lax.axis_index("s") * _NC + lax.axis_index("c")
    nch = T // C
    bufs = (buf0, buf1)
    sems = (sem0, sem1)

    def dma(i):
        return pltpu.make_async_copy(
            x_hbm.at[wid, pl.ds(i * C, C)], bufs[i % 2], sems[i % 2])

    dma(0).start()
    for i in range(nch):
        if i + 1 < nch:
            dma(i + 1).start()
        dma(i).wait()
        buf = bufs[i % 2]
        t0 = i * C

        def t_body(t, _):
            for c in range(S // _L):
                acc = buf[t, 0, pl.ds(c * _L, _L)]
                for k in range(1, K):
                    acc = acc + buf[t, k, pl.ds(c * _L, _L)]
                mbuf[t0 + t, pl.ds(c * _L, _L)] = acc * (1.0 / K)
            return 0

        lax.fori_loop(0, C, t_body, 0)

    pltpu.sync_copy(mbuf, m_hbm.at[wid])


def _tc_proj(m_ref, w_ref, b_ref, o_ref, *, K):
    mb = m_ref[...]
    BB, T, S = mb.shape
    m = mb.reshape(BB * T, S)
    z = jnp.dot(m, w_ref[...], preferred_element_type=jnp.float32)
    z = z + b_ref[...]
    zt = jnp.tile(z, (1, K))
    o_ref[...] = zt.reshape(BB, T, K * z.shape[1])


def kernel(x, W, b):
    B, T, K, S = x.shape
    Z = W.shape[1]
    C = 10

    mesh = plsc.VectorSubcoreMesh(core_axis_name="c", subcore_axis_name="s")
    sc_mean = pl.kernel(
        functools.partial(_sc_mean, T=T, K=K, S=S, C=C),
        out_type=jax.ShapeDtypeStruct((B, T, S), jnp.float32),
        mesh=mesh,
        scratch_types=[
            pltpu.VMEM((C, K, S), jnp.float32),
            pltpu.VMEM((C, K, S), jnp.float32),
            pltpu.VMEM((T, S), jnp.float32),
            pltpu.SemaphoreType.DMA,
            pltpu.SemaphoreType.DMA,
        ],
    )
    m = sc_mean(x)

    BB = 8
    out = pl.pallas_call(
        functools.partial(_tc_proj, K=K),
        grid=(B // BB,),
        in_specs=[
            pl.BlockSpec((BB, T, S), lambda i: (i, 0, 0)),
            pl.BlockSpec((S, Z), lambda i: (0, 0)),
            pl.BlockSpec((1, Z), lambda i: (0, 0)),
        ],
        out_specs=pl.BlockSpec((BB, T, K * Z), lambda i: (i, 0, 0)),
        out_shape=jax.ShapeDtypeStruct((B, T, K * Z), jnp.float32),
        compiler_params=pltpu.CompilerParams(
            dimension_semantics=("parallel",)),
    )(m, W, b.reshape(1, Z))
    return out.reshape(B, T, K, Z)


# speedup vs baseline: 2.0439x; 1.6228x over previous
import functools

import jax
import jax.numpy as jnp
from jax.experimental import pallas as pl
from jax.experimental.pallas import tpu as pltpu


def _enc_block(x_ref, w_ref, b_ref, o_ref, *, K):
    xs = x_ref[...]
    BB, TT, _, S = xs.shape
    m = jnp.sum(xs, axis=2) * (1.0 / K)
    m2 = m.reshape(BB * TT, S)
    z = jnp.dot(m2, w_ref[...], preferred_element_type=jnp.float32)
    z = z + b_ref[...]
    zt = jnp.tile(z, (1, K))
    o_ref[...] = zt.reshape(BB, TT, K * z.shape[1])


def kernel(x, W, b):
    B, T, K, S = x.shape
    Z = W.shape[1]
    BB = 4
    out = pl.pallas_call(
        functools.partial(_enc_block, K=K),
        grid=(B // BB,),
        in_specs=[
            pl.BlockSpec((BB, T, K, S), lambda i: (i, 0, 0, 0)),
            pl.BlockSpec((S, Z), lambda i: (0, 0)),
            pl.BlockSpec((1, Z), lambda i: (0, 0)),
        ],
        out_specs=pl.BlockSpec((BB, T, K * Z), lambda i: (i, 0, 0)),
        out_shape=jax.ShapeDtypeStruct((B, T, K * Z), jnp.float32),
        compiler_params=pltpu.CompilerParams(
            dimension_semantics=("arbitrary",)),
    )(x, W, b.reshape(1, Z))
    return out.reshape(B, T, K, Z)


# speedup vs baseline: 2.0609x; 1.0083x over previous
import functools

import jax
import jax.numpy as jnp
from jax.experimental import pallas as pl
from jax.experimental.pallas import tpu as pltpu


def _enc_block(x_ref, w_ref, b_ref, o_ref, *, K):
    xs = x_ref[...]
    BB, TT, _, S = xs.shape
    m = jnp.sum(xs, axis=2) * (1.0 / K)
    m2 = m.reshape(BB * TT, S)
    z = jnp.dot(m2, w_ref[...], preferred_element_type=jnp.float32)
    z = z + b_ref[...]
    zt = jnp.tile(z, (1, K))
    o_ref[...] = zt.reshape(BB, TT, K * z.shape[1])


def kernel(x, W, b):
    B, T, K, S = x.shape
    Z = W.shape[1]
    BB = 8
    out = pl.pallas_call(
        functools.partial(_enc_block, K=K),
        grid=(B // BB,),
        in_specs=[
            pl.BlockSpec((BB, T, K, S), lambda i: (i, 0, 0, 0)),
            pl.BlockSpec((S, Z), lambda i: (0, 0)),
            pl.BlockSpec((1, Z), lambda i: (0, 0)),
        ],
        out_specs=pl.BlockSpec((BB, T, K * Z), lambda i: (i, 0, 0)),
        out_shape=jax.ShapeDtypeStruct((B, T, K * Z), jnp.float32),
        compiler_params=pltpu.CompilerParams(
            dimension_semantics=("arbitrary",)),
    )(x, W, b.reshape(1, Z))
    return out.reshape(B, T, K, Z)
